# baseline (speedup 1.0000x reference)
import jax
import jax.numpy as jnp
from jax.experimental import pallas as pl
from jax.experimental.pallas import tpu as pltpu


_Q = 4
_TBQ = 128


def _tail(diffT, K):
    dm = diffT.reshape(diffT.size // 128, 128)
    lane = jax.lax.broadcasted_iota(jnp.int32, dm.shape, 1)
    left = lane < K
    neg = jnp.float32(-3.0e38)
    ml = jnp.max(jnp.where(left, dm, neg), axis=1, keepdims=True)
    mr = jnp.max(jnp.where(left, neg, dm), axis=1, keepdims=True)
    need = jnp.where(left, ml, mr) <= 0.0
    first = (lane == 0) | (lane == K)
    dec = jnp.where((dm > 0.0) | (first & need), 1.0, 0.0)
    keep = jax.nn.sigmoid(dm)
    return dec, keep


def _make_body(nq):
    def _body(*refs):
        x_refs = refs[:nq]
        w1_ref, b1_ref, w2_ref, b2d_ref, dec_ref, keep_ref = refs[nq:]
        for q, x_ref in enumerate(x_refs):
            TB, K, D = x_ref.shape
            N = TB * K
            R = N // 128
            x2 = x_ref[...].reshape(N, D)
            hT = jnp.maximum(
                jax.lax.dot_general(
                    w1_ref[...], x2, (((0,), (1,)), ((), ())),
                    preferred_element_type=jnp.float32,
                )
                + b1_ref[...].reshape(D // 2, 1),
                0.0,
            )
            logitsT = jax.lax.dot_general(
                w2_ref[...], hT, (((0,), (0,)), ((), ())),
                preferred_element_type=jnp.float32,
            )
            diffT = logitsT[1:2, :] - logitsT[0:1, :] + b2d_ref[0, 0]
            dec, keep = _tail(diffT, K)
            dec_ref[q * R : (q + 1) * R, :] = dec
            keep_ref[q * R : (q + 1) * R, :] = keep

    return _body


def kernel(slots, W1, b1, W2, b2):
    B, K, D = slots.shape
    F = W1.shape[1]
    if B % (_Q * _TBQ) == 0:
        nq, tbq = _Q, _TBQ
    else:
        nq, tbq = 1, min(_TBQ, B)
    TB = nq * tbq
    grid = (B // TB,)
    b2d = (b2[1] - b2[0]).reshape(1, 1)
    x_specs = [
        pl.BlockSpec((tbq, K, D), lambda i, q=q: (nq * i + q, 0, 0))
        for q in range(nq)
    ]
    dec, keep = pl.pallas_call(
        _make_body(nq),
        grid=grid,
        in_specs=x_specs
        + [
            pl.BlockSpec((D, F), lambda i: (0, 0)),
            pl.BlockSpec((F,), lambda i: (0,)),
            pl.BlockSpec((F, 2), lambda i: (0, 0)),
            pl.BlockSpec(memory_space=pltpu.SMEM),
        ],
        out_specs=[
            pl.BlockSpec((TB * K // 128, 128), lambda i: (i, 0)),
            pl.BlockSpec((TB * K // 128, 128), lambda i: (i, 0)),
        ],
        out_shape=[
            jax.ShapeDtypeStruct((B * K // 128, 128), jnp.float32),
            jax.ShapeDtypeStruct((B * K // 128, 128), jnp.float32),
        ],
        compiler_params=pltpu.CompilerParams(
            dimension_semantics=("parallel",),
        ),
    )(*([slots] * nq), W1, b1, W2, b2d)
    return (dec.reshape(B, K), keep.reshape(B, K))


# speedup vs baseline: 1.0011x; 1.0011x over previous
import jax
import jax.numpy as jnp
from jax.experimental import pallas as pl
from jax.experimental.pallas import tpu as pltpu


def _tail(diffT, K):
    dm = diffT.reshape(diffT.size // 128, 128)
    lane = jax.lax.broadcasted_iota(jnp.int32, dm.shape, 1)
    left = lane < K
    neg = jnp.float32(-3.0e38)
    ml = jnp.max(jnp.where(left, dm, neg), axis=1, keepdims=True)
    mr = jnp.max(jnp.where(left, neg, dm), axis=1, keepdims=True)
    need = jnp.where(left, ml, mr) <= 0.0
    first = (lane == 0) | (lane == K)
    dec = jnp.where((dm > 0.0) | (first & need), 1.0, 0.0)
    keep = jax.nn.sigmoid(dm)
    return dec, keep


def _make_body(nbuf, nchunks, tbc):
    def _body(x_hbm, w1_ref, b1_ref, w2_ref, b2d_ref, dec_ref, keep_ref,
              buf, sem):
        _, K, D = x_hbm.shape
        R = tbc * K // 128

        def start(c, s):
            pltpu.make_async_copy(
                x_hbm.at[pl.ds(c * tbc, tbc)], buf.at[s], sem.at[s]
            ).start()

        def wait(s):
            pltpu.make_async_copy(
                x_hbm.at[pl.ds(0, tbc)], buf.at[s], sem.at[s]
            ).wait()

        for s in range(nbuf):
            start(s, s)

        def round_(r, carry):
            for s in range(nbuf):
                c = r * nbuf + s
                wait(s)
                x2 = buf[s].reshape(tbc * K, D)
                hT = jnp.maximum(
                    jax.lax.dot_general(
                        w1_ref[...], x2, (((0,), (1,)), ((), ())),
                        preferred_element_type=jnp.float32,
                    )
                    + b1_ref[...].reshape(D // 2, 1),
                    0.0,
                )
                logitsT = jax.lax.dot_general(
                    w2_ref[...], hT, (((0,), (0,)), ((), ())),
                    preferred_element_type=jnp.float32,
                )
                diffT = logitsT[1:2, :] - logitsT[0:1, :] + b2d_ref[0, 0]
                dec, keep = _tail(diffT, K)
                row = c * R
                dec_ref[pl.ds(row, R), :] = dec
                keep_ref[pl.ds(row, R), :] = keep
                nc = c + nbuf

                @pl.when(nc < nchunks)
                def _():
                    start(nc, s)

            return carry

        jax.lax.fori_loop(0, nchunks // nbuf, round_, 0)

    return _body


def kernel(slots, W1, b1, W2, b2):
    B, K, D = slots.shape
    F = W1.shape[1]
    tbc = min(128, B)
    nchunks = B // tbc
    nbuf = 8 if nchunks % 8 == 0 else 1
    b2d = (b2[1] - b2[0]).reshape(1, 1)
    dec, keep = pl.pallas_call(
        _make_body(nbuf, nchunks, tbc),
        in_specs=[
            pl.BlockSpec(memory_space=pl.ANY),
            pl.BlockSpec(memory_space=pltpu.VMEM),
            pl.BlockSpec(memory_space=pltpu.VMEM),
            pl.BlockSpec(memory_space=pltpu.VMEM),
            pl.BlockSpec(memory_space=pltpu.SMEM),
        ],
        out_specs=[
            pl.BlockSpec(memory_space=pltpu.VMEM),
            pl.BlockSpec(memory_space=pltpu.VMEM),
        ],
        out_shape=[
            jax.ShapeDtypeStruct((B * K // 128, 128), jnp.float32),
            jax.ShapeDtypeStruct((B * K // 128, 128), jnp.float32),
        ],
        scratch_shapes=[
            pltpu.VMEM((nbuf, tbc, K, D), jnp.float32),
            pltpu.SemaphoreType.DMA((nbuf,)),
        ],
    )(slots, W1, b1, W2, b2d)
    return (dec.reshape(B, K), keep.reshape(B, K))
